# baseline (speedup 1.0000x reference)
import functools

import jax
import jax.numpy as jnp
from jax import lax
from jax.experimental import pallas as pl
from jax.experimental.pallas import tpu as pltpu
from jax.experimental.pallas import tpu_sc as plsc

_BETA = 0.25
_N_TOK = 8192
_DIM = 256
_K = 8192

_TM = 256
_TN = 512

_SC_NC = 2
_SC_NS = 16
_SC_NW = _SC_NC * _SC_NS
_ROWS_PER_W = _N_TOK // _SC_NW
_IDX_CHUNK = 128


def _vq_argmin_body(x2_ref, z_ref, e_ref, idx_ref, lsum_ref, bval, bidx):
    i = pl.program_id(0)
    j = pl.program_id(1)
    nj = pl.num_programs(1)

    z = z_ref[...]
    e = e_ref[...]
    c = lax.dot_general(z, e, (((1,), (1,)), ((), ())))
    e2 = jnp.sum(e * e, axis=1)
    t = x2_ref[...] - 2.0 * c
    d2 = jnp.maximum(t + e2[None, :], 0.0)
    dist = jnp.sqrt(d2)

    tmin = jnp.min(dist, axis=1, keepdims=True)
    cols = lax.broadcasted_iota(jnp.int32, (_TM, _TN), 1) + j * _TN
    big = jnp.int32(jnp.iinfo(jnp.int32).max)
    targ = jnp.min(jnp.where(dist == tmin, cols, big), axis=1,
                   keepdims=True)

    @pl.when(j == 0)
    def _():
        bval[...] = tmin
        bidx[...] = targ

    @pl.when(j > 0)
    def _():
        better = tmin < bval[...]
        bval[...] = jnp.where(better, tmin, bval[...])
        bidx[...] = jnp.where(better, targ, bidx[...])

    @pl.when(j == nj - 1)
    def _():
        idx_ref[...] = bidx[...]
        part = jnp.sum(bval[...] ** 2, axis=0, keepdims=True)
        prev = jnp.where(i == 0, jnp.zeros_like(part), lsum_ref[...])
        lsum_ref[...] = prev + part


def _vq_argmin(x2, z_flat, embedding_weight):
    grid = (_N_TOK // _TM, _K // _TN)
    return pl.pallas_call(
        _vq_argmin_body,
        grid=grid,
        in_specs=[
            pl.BlockSpec((_TM, 1), lambda i, j: (i, 0)),
            pl.BlockSpec((_TM, _DIM), lambda i, j: (i, 0)),
            pl.BlockSpec((_TN, _DIM), lambda i, j: (j, 0)),
        ],
        out_specs=[
            pl.BlockSpec((_TM, 1), lambda i, j: (i, 0)),
            pl.BlockSpec((1, 1), lambda i, j: (0, 0)),
        ],
        out_shape=[
            jax.ShapeDtypeStruct((_N_TOK, 1), jnp.int32),
            jax.ShapeDtypeStruct((1, 1), jnp.float32),
        ],
        scratch_shapes=[
            pltpu.VMEM((_TM, 1), jnp.float32),
            pltpu.VMEM((_TM, 1), jnp.int32),
        ],
        compiler_params=pltpu.CompilerParams(
            dimension_semantics=("arbitrary", "arbitrary")),
    )(x2, z_flat, embedding_weight)


def _sc_gather_body(table_hbm, idx_hbm, out_hbm, idx_v, rows_v, sem):
    c = lax.axis_index("c")
    s = lax.axis_index("s")
    wid = s * _SC_NC + c
    base = wid * _ROWS_PER_W
    nchunk = _ROWS_PER_W // _IDX_CHUNK
    pltpu.sync_copy(idx_hbm.at[pl.ds(wid * nchunk, nchunk)], idx_v)
    for k in range(nchunk):
        pltpu.async_copy(
            table_hbm.at[idx_v.at[k]],
            rows_v.at[pl.ds(k * _IDX_CHUNK, _IDX_CHUNK)],
            sem,
        ).wait()
    pltpu.sync_copy(rows_v, out_hbm.at[pl.ds(base, _ROWS_PER_W)])


def _sc_gather(embedding_weight, idx2d):
    mesh = plsc.VectorSubcoreMesh(core_axis_name="c", subcore_axis_name="s")
    nchunk = _ROWS_PER_W // _IDX_CHUNK
    call = functools.partial(
        pl.kernel,
        mesh=mesh,
        out_type=jax.ShapeDtypeStruct((_N_TOK, _DIM), jnp.float32),
        scratch_types=[
            pltpu.VMEM((nchunk, _IDX_CHUNK), jnp.int32),
            pltpu.VMEM((_ROWS_PER_W, _DIM), jnp.float32),
            pltpu.SemaphoreType.DMA,
        ],
    )(_sc_gather_body)
    return call(embedding_weight, idx2d)


def kernel(z, embedding_weight):
    zp = jnp.transpose(z, (0, 2, 3, 1))
    z_shape = zp.shape
    z_flat = zp.reshape(-1, embedding_weight.shape[1])
    x2 = jnp.sum(z_flat ** 2, axis=1, keepdims=True)

    idx_col, lsum = _vq_argmin(x2, z_flat, embedding_weight)
    nearest_embs = idx_col.reshape(-1)

    z_q_flat = _sc_gather(embedding_weight,
                          nearest_embs.reshape(-1, _IDX_CHUNK))
    z_q = z_q_flat.reshape(z_shape)

    m = lsum[0, 0] / jnp.float32(_N_TOK * _DIM)
    loss = m + _BETA * m

    z_q_st = zp + (z_q - zp)
    z_q_out = jnp.transpose(z_q_st, (0, 3, 1, 2))
    return (z_q_out, loss, (nearest_embs, z_flat))


# speedup vs baseline: 1.6396x; 1.6396x over previous
import functools

import jax
import jax.numpy as jnp
from jax import lax
from jax.experimental import pallas as pl
from jax.experimental.pallas import tpu as pltpu
from jax.experimental.pallas import tpu_sc as plsc

_BETA = 0.25
_N_TOK = 8192
_DIM = 256
_K = 8192

_TM = 256
_TN = 512

_SC_NC = 2
_SC_NS = 16
_SC_NW = _SC_NC * _SC_NS
_ROWS_PER_W = _N_TOK // _SC_NW
_IDX_CHUNK = 128


def _vq_argmin_body(x2_ref, z_ref, e_ref, idx_ref, lsum_ref):
    i = pl.program_id(0)

    z = z_ref[...]
    e = e_ref[...]
    best_v = None
    best_i = None
    for j in range(_K // _TN):
        ej = e[j * _TN:(j + 1) * _TN, :]
        c = lax.dot_general(z, ej, (((1,), (1,)), ((), ())))
        e2 = jnp.sum(ej * ej, axis=1)
        t = x2_ref[...] - 2.0 * c
        d2 = jnp.maximum(t + e2[None, :], 0.0)
        dist = jnp.sqrt(d2)

        tmin = jnp.min(dist, axis=1, keepdims=True)
        cols = lax.broadcasted_iota(jnp.int32, (_TM, _TN), 1) + j * _TN
        big = jnp.int32(jnp.iinfo(jnp.int32).max)
        targ = jnp.min(jnp.where(dist == tmin, cols, big), axis=1,
                       keepdims=True)
        if best_v is None:
            best_v, best_i = tmin, targ
        else:
            better = tmin < best_v
            best_v = jnp.where(better, tmin, best_v)
            best_i = jnp.where(better, targ, best_i)

    idx_ref[...] = best_i
    part = jnp.sum(best_v ** 2, axis=0, keepdims=True)
    prev = jnp.where(i == 0, jnp.zeros_like(part), lsum_ref[...])
    lsum_ref[...] = prev + part


def _vq_argmin(x2, z_flat, embedding_weight):
    grid = (_N_TOK // _TM,)
    return pl.pallas_call(
        _vq_argmin_body,
        grid=grid,
        in_specs=[
            pl.BlockSpec((_TM, 1), lambda i: (i, 0)),
            pl.BlockSpec((_TM, _DIM), lambda i: (i, 0)),
            pl.BlockSpec((_K, _DIM), lambda i: (0, 0)),
        ],
        out_specs=[
            pl.BlockSpec((_TM, 1), lambda i: (i, 0)),
            pl.BlockSpec((1, 1), lambda i: (0, 0)),
        ],
        out_shape=[
            jax.ShapeDtypeStruct((_N_TOK, 1), jnp.int32),
            jax.ShapeDtypeStruct((1, 1), jnp.float32),
        ],
        compiler_params=pltpu.CompilerParams(
            dimension_semantics=("arbitrary",)),
    )(x2, z_flat, embedding_weight)


def _sc_gather_body(table_hbm, idx_hbm, out_hbm, idx_v, rows_v, sem):
    c = lax.axis_index("c")
    s = lax.axis_index("s")
    wid = s * _SC_NC + c
    base = wid * _ROWS_PER_W
    nchunk = _ROWS_PER_W // _IDX_CHUNK
    pltpu.sync_copy(idx_hbm.at[pl.ds(wid * nchunk, nchunk)], idx_v)
    for k in range(nchunk):
        pltpu.async_copy(
            table_hbm.at[idx_v.at[k]],
            rows_v.at[pl.ds(k * _IDX_CHUNK, _IDX_CHUNK)],
            sem,
        ).wait()
    pltpu.sync_copy(rows_v, out_hbm.at[pl.ds(base, _ROWS_PER_W)])


def _sc_gather(embedding_weight, idx2d):
    mesh = plsc.VectorSubcoreMesh(core_axis_name="c", subcore_axis_name="s")
    nchunk = _ROWS_PER_W // _IDX_CHUNK
    call = functools.partial(
        pl.kernel,
        mesh=mesh,
        out_type=jax.ShapeDtypeStruct((_N_TOK, _DIM), jnp.float32),
        scratch_types=[
            pltpu.VMEM((nchunk, _IDX_CHUNK), jnp.int32),
            pltpu.VMEM((_ROWS_PER_W, _DIM), jnp.float32),
            pltpu.SemaphoreType.DMA,
        ],
    )(_sc_gather_body)
    return call(embedding_weight, idx2d)


def kernel(z, embedding_weight):
    zp = jnp.transpose(z, (0, 2, 3, 1))
    z_shape = zp.shape
    z_flat = zp.reshape(-1, embedding_weight.shape[1])
    x2 = jnp.sum(z_flat ** 2, axis=1, keepdims=True)

    idx_col, lsum = _vq_argmin(x2, z_flat, embedding_weight)
    nearest_embs = idx_col.reshape(-1)

    z_q_flat = _sc_gather(embedding_weight,
                          nearest_embs.reshape(-1, _IDX_CHUNK))
    z_q = z_q_flat.reshape(z_shape)

    m = lsum[0, 0] / jnp.float32(_N_TOK * _DIM)
    loss = m + _BETA * m

    z_q_st = zp + (z_q - zp)
    z_q_out = jnp.transpose(z_q_st, (0, 3, 1, 2))
    return (z_q_out, loss, (nearest_embs, z_flat))


# speedup vs baseline: 2.7050x; 1.6497x over previous
import functools

import jax
import jax.numpy as jnp
from jax import lax
from jax.experimental import pallas as pl
from jax.experimental.pallas import tpu as pltpu
from jax.experimental.pallas import tpu_sc as plsc

_BETA = 0.25
_N_TOK = 8192
_DIM = 256
_K = 8192

_TM = 128
_TN = 1024

_SC_NC = 2
_SC_NS = 16
_SC_NW = _SC_NC * _SC_NS
_ROWS_PER_W = _N_TOK // _SC_NW
_IDX_CHUNK = 128


def _vq_argmin_body(x2_ref, z_ref, e_ref, idx_ref, lsum_ref, e2_ref,
                    lacc_ref):
    i = pl.program_id(0)

    @pl.when(i == 0)
    def _():
        for j in range(_K // _TN):
            ej = e_ref[j * _TN:(j + 1) * _TN, :]
            e2c = jnp.sum(ej * ej, axis=1, keepdims=True)
            e2_ref[j * _TN:(j + 1) * _TN, :] = jnp.broadcast_to(
                e2c, (_TN, _TM))

    z = z_ref[...]
    x2 = x2_ref[...]
    nsub = 8
    acc_v = None
    acc_i = None
    for j in range(_K // _TN):
        ej = e_ref[j * _TN:(j + 1) * _TN, :]
        c = lax.dot_general(ej, z, (((1,), (1,)), ((), ())))
        t = x2 - 2.0 * c
        d2 = jnp.maximum(t + e2_ref[j * _TN:(j + 1) * _TN, :], 0.0)
        dist = jnp.sqrt(d2)
        rows = lax.broadcasted_iota(jnp.int32, (_TN, _TM), 0) + j * _TN
        for r in range(_TN // nsub):
            dv = dist[r * nsub:(r + 1) * nsub, :]
            iv = rows[r * nsub:(r + 1) * nsub, :]
            if acc_v is None:
                acc_v, acc_i = dv, iv
            else:
                take = dv < acc_v
                acc_v = jnp.where(take, dv, acc_v)
                acc_i = jnp.where(take, iv, acc_i)

    h = nsub // 2
    while h >= 1:
        va, vb = acc_v[:h, :], acc_v[h:2 * h, :]
        ia, ib = acc_i[:h, :], acc_i[h:2 * h, :]
        take = (vb < va) | ((vb == va) & (ib < ia))
        acc_v = jnp.where(take, vb, va)
        acc_i = jnp.where(take, ib, ia)
        h //= 2

    idx_ref[...] = acc_i
    part = acc_v * acc_v
    prev = jnp.where(i == 0, jnp.zeros_like(part), lacc_ref[...])
    lacc_ref[...] = prev + part

    @pl.when(i == pl.num_programs(0) - 1)
    def _():
        lsum_ref[...] = jnp.sum(lacc_ref[...], axis=1, keepdims=True)


def _vq_argmin(x2row, z_flat, embedding_weight):
    grid = (_N_TOK // _TM,)
    return pl.pallas_call(
        _vq_argmin_body,
        grid=grid,
        in_specs=[
            pl.BlockSpec((1, _TM), lambda i: (0, i)),
            pl.BlockSpec((_TM, _DIM), lambda i: (i, 0)),
            pl.BlockSpec((_K, _DIM), lambda i: (0, 0)),
        ],
        out_specs=[
            pl.BlockSpec((1, _TM), lambda i: (0, i)),
            pl.BlockSpec((1, 1), lambda i: (0, 0)),
        ],
        out_shape=[
            jax.ShapeDtypeStruct((1, _N_TOK), jnp.int32),
            jax.ShapeDtypeStruct((1, 1), jnp.float32),
        ],
        scratch_shapes=[
            pltpu.VMEM((_K, _TM), jnp.float32),
            pltpu.VMEM((1, _TM), jnp.float32),
        ],
        compiler_params=pltpu.CompilerParams(
            dimension_semantics=("arbitrary",)),
    )(x2row, z_flat, embedding_weight)


def _sc_gather_body(table_hbm, idx_hbm, out_hbm, idx_v, rows_v, sem):
    c = lax.axis_index("c")
    s = lax.axis_index("s")
    wid = s * _SC_NC + c
    base = wid * _ROWS_PER_W
    nchunk = _ROWS_PER_W // _IDX_CHUNK
    pltpu.sync_copy(idx_hbm.at[pl.ds(wid * nchunk, nchunk)], idx_v)
    for k in range(nchunk):
        pltpu.async_copy(
            table_hbm.at[idx_v.at[k]],
            rows_v.at[pl.ds(k * _IDX_CHUNK, _IDX_CHUNK)],
            sem,
        ).wait()
    pltpu.sync_copy(rows_v, out_hbm.at[pl.ds(base, _ROWS_PER_W)])


def _sc_gather(embedding_weight, idx2d):
    mesh = plsc.VectorSubcoreMesh(core_axis_name="c", subcore_axis_name="s")
    nchunk = _ROWS_PER_W // _IDX_CHUNK
    call = functools.partial(
        pl.kernel,
        mesh=mesh,
        out_type=jax.ShapeDtypeStruct((_N_TOK, _DIM), jnp.float32),
        scratch_types=[
            pltpu.VMEM((nchunk, _IDX_CHUNK), jnp.int32),
            pltpu.VMEM((_ROWS_PER_W, _DIM), jnp.float32),
            pltpu.SemaphoreType.DMA,
        ],
    )(_sc_gather_body)
    return call(embedding_weight, idx2d)


def kernel(z, embedding_weight):
    zp = jnp.transpose(z, (0, 2, 3, 1))
    z_shape = zp.shape
    z_flat = zp.reshape(-1, embedding_weight.shape[1])
    x2row = jnp.sum(z_flat ** 2, axis=1)[None, :]

    idx_row, lsum = _vq_argmin(x2row, z_flat, embedding_weight)
    nearest_embs = idx_row.reshape(-1)

    z_q_flat = _sc_gather(embedding_weight,
                          nearest_embs.reshape(-1, _IDX_CHUNK))
    z_q = z_q_flat.reshape(z_shape)

    m = lsum[0, 0] / jnp.float32(_N_TOK * _DIM)
    loss = m + _BETA * m

    z_q_st = zp + (z_q - zp)
    z_q_out = jnp.transpose(z_q_st, (0, 3, 1, 2))
    return (z_q_out, loss, (nearest_embs, z_flat))


# speedup vs baseline: 2.7775x; 1.0268x over previous
import functools

import jax
import jax.numpy as jnp
from jax import lax
from jax.experimental import pallas as pl
from jax.experimental.pallas import tpu as pltpu
from jax.experimental.pallas import tpu_sc as plsc

_BETA = 0.25
_N_TOK = 8192
_DIM = 256
_K = 8192

_TM = 256
_TN = 1024

_SC_NC = 2
_SC_NS = 16
_SC_NW = _SC_NC * _SC_NS
_ROWS_PER_W = _N_TOK // _SC_NW
_IDX_CHUNK = 128


def _vq_argmin_body(x2_ref, z_ref, e_ref, idx_ref, lsum_ref, e2_ref,
                    lacc_ref):
    i = pl.program_id(0)

    @pl.when(i == 0)
    def _():
        for j in range(_K // _TN):
            ej = e_ref[j * _TN:(j + 1) * _TN, :]
            e2c = jnp.sum(ej * ej, axis=1, keepdims=True)
            e2_ref[j * _TN:(j + 1) * _TN, :] = jnp.broadcast_to(
                e2c, (_TN, _TM))

    z = z_ref[...]
    x2 = x2_ref[...]
    nsub = 8
    acc_v = None
    acc_i = None
    for j in range(_K // _TN):
        ej = e_ref[j * _TN:(j + 1) * _TN, :]
        c = lax.dot_general(ej, z, (((1,), (1,)), ((), ())))
        t = x2 - 2.0 * c
        d2 = jnp.maximum(t + e2_ref[j * _TN:(j + 1) * _TN, :], 0.0)
        dist = jnp.sqrt(d2)
        rows = lax.broadcasted_iota(jnp.int32, (_TN, _TM), 0) + j * _TN
        for r in range(_TN // nsub):
            dv = dist[r * nsub:(r + 1) * nsub, :]
            iv = rows[r * nsub:(r + 1) * nsub, :]
            if acc_v is None:
                acc_v, acc_i = dv, iv
            else:
                take = dv < acc_v
                acc_v = jnp.where(take, dv, acc_v)
                acc_i = jnp.where(take, iv, acc_i)

    h = nsub // 2
    while h >= 1:
        va, vb = acc_v[:h, :], acc_v[h:2 * h, :]
        ia, ib = acc_i[:h, :], acc_i[h:2 * h, :]
        take = (vb < va) | ((vb == va) & (ib < ia))
        acc_v = jnp.where(take, vb, va)
        acc_i = jnp.where(take, ib, ia)
        h //= 2

    idx_ref[...] = acc_i
    part = acc_v * acc_v
    prev = jnp.where(i == 0, jnp.zeros_like(part), lacc_ref[...])
    lacc_ref[...] = prev + part

    @pl.when(i == pl.num_programs(0) - 1)
    def _():
        lsum_ref[...] = jnp.sum(lacc_ref[...], axis=1, keepdims=True)


def _vq_argmin(x2row, z_flat, embedding_weight):
    grid = (_N_TOK // _TM,)
    return pl.pallas_call(
        _vq_argmin_body,
        grid=grid,
        in_specs=[
            pl.BlockSpec((1, _TM), lambda i: (0, i)),
            pl.BlockSpec((_TM, _DIM), lambda i: (i, 0)),
            pl.BlockSpec((_K, _DIM), lambda i: (0, 0)),
        ],
        out_specs=[
            pl.BlockSpec((1, _TM), lambda i: (0, i)),
            pl.BlockSpec((1, 1), lambda i: (0, 0)),
        ],
        out_shape=[
            jax.ShapeDtypeStruct((1, _N_TOK), jnp.int32),
            jax.ShapeDtypeStruct((1, 1), jnp.float32),
        ],
        scratch_shapes=[
            pltpu.VMEM((_K, _TM), jnp.float32),
            pltpu.VMEM((1, _TM), jnp.float32),
        ],
        compiler_params=pltpu.CompilerParams(
            dimension_semantics=("arbitrary",)),
    )(x2row, z_flat, embedding_weight)


def _sc_gather_body(table_hbm, idx_hbm, out_hbm, idx_v, rows_v, sem):
    c = lax.axis_index("c")
    s = lax.axis_index("s")
    wid = s * _SC_NC + c
    base = wid * _ROWS_PER_W
    nchunk = _ROWS_PER_W // _IDX_CHUNK
    pltpu.sync_copy(idx_hbm.at[pl.ds(wid * nchunk, nchunk)], idx_v)
    for k in range(nchunk):
        pltpu.async_copy(
            table_hbm.at[idx_v.at[k]],
            rows_v.at[pl.ds(k * _IDX_CHUNK, _IDX_CHUNK)],
            sem,
        ).wait()
    pltpu.sync_copy(rows_v, out_hbm.at[pl.ds(base, _ROWS_PER_W)])


def _sc_gather(embedding_weight, idx2d):
    mesh = plsc.VectorSubcoreMesh(core_axis_name="c", subcore_axis_name="s")
    nchunk = _ROWS_PER_W // _IDX_CHUNK
    call = functools.partial(
        pl.kernel,
        mesh=mesh,
        out_type=jax.ShapeDtypeStruct((_N_TOK, _DIM), jnp.float32),
        scratch_types=[
            pltpu.VMEM((nchunk, _IDX_CHUNK), jnp.int32),
            pltpu.VMEM((_ROWS_PER_W, _DIM), jnp.float32),
            pltpu.SemaphoreType.DMA,
        ],
    )(_sc_gather_body)
    return call(embedding_weight, idx2d)


def kernel(z, embedding_weight):
    zp = jnp.transpose(z, (0, 2, 3, 1))
    z_shape = zp.shape
    z_flat = zp.reshape(-1, embedding_weight.shape[1])
    x2row = jnp.sum(z_flat ** 2, axis=1)[None, :]

    idx_row, lsum = _vq_argmin(x2row, z_flat, embedding_weight)
    nearest_embs = idx_row.reshape(-1)

    gather_idx = jnp.clip(nearest_embs, 0, _K - 1)
    z_q_flat = _sc_gather(embedding_weight,
                          gather_idx.reshape(-1, _IDX_CHUNK))
    z_q = z_q_flat.reshape(z_shape)

    m = lsum[0, 0] / jnp.float32(_N_TOK * _DIM)
    loss = m + _BETA * m

    z_q_st = zp + (z_q - zp)
    z_q_out = jnp.transpose(z_q_st, (0, 3, 1, 2))
    return (z_q_out, loss, (nearest_embs, z_flat))
